# Initial kernel scaffold; baseline (speedup 1.0000x reference)
import jax
import jax.numpy as jnp
from jax.experimental import pallas as pl


def kernel(x, edge_index, W1, b1, W2, b2):
    raise NotImplementedError("write your pallas kernel here")


# speedup vs baseline: 10.6224x; 10.6224x over previous
import functools

import jax
import jax.numpy as jnp
from jax import lax
from jax.experimental import pallas as pl
from jax.experimental.pallas import tpu as pltpu
from jax.experimental.pallas import tpu_sc as plsc

_NN = 10000
_NP = 10240
_E = 160000
_W = 16


def _vmesh():
    return plsc.VectorSubcoreMesh(core_axis_name="c", subcore_axis_name="s")


def _sc_degree(dst, zeros640, ones128):
    @functools.partial(
        pl.kernel,
        out_type=jax.ShapeDtypeStruct((2, _NP, 128), jnp.float32),
        mesh=_vmesh(),
        scratch_types=[
            pltpu.VMEM_SHARED((_NP, 128), jnp.float32),
            pltpu.VMEM((128, 128), jnp.float32),
            pltpu.VMEM((128,), jnp.int32),
            pltpu.VMEM((8,), jnp.int32),
        ],
    )
    def k(dst_hbm, zeros_hbm, ones_hbm, out_hbm, acc, ones_v, idx_v, idx8_v):
        c = lax.axis_index("c")
        s = lax.axis_index("s")

        pltpu.sync_copy(ones_hbm, ones_v)
        pltpu.sync_copy(zeros_hbm, acc.at[pl.ds(s * 640, 640)])

        plsc.subcore_barrier()

        base = c * 80000 + s * 5000

        @pl.loop(0, 39)
        def _(j):
            pltpu.sync_copy(dst_hbm.at[pl.ds(base + j * 128, 128)], idx_v)
            pltpu.sync_copy(ones_v, acc.at[idx_v], add=True)

        pltpu.sync_copy(dst_hbm.at[pl.ds(base + 4992, 8)], idx8_v)
        pltpu.sync_copy(ones_v.at[pl.ds(0, 8)], acc.at[idx8_v], add=True)

        plsc.subcore_barrier()
        pltpu.sync_copy(acc.at[pl.ds(s * 640, 640)],
                        out_hbm.at[c, pl.ds(s * 640, 640)])

    return k(dst, zeros640, ones128)


def _sc_agg(g, src, dst, fsplit):
    n_chunks, tail = (78, 16) if fsplit else (39, 8)

    @functools.partial(
        pl.kernel,
        out_type=jax.ShapeDtypeStruct((2, _NN, 128), jnp.float32),
        mesh=_vmesh(),
        scratch_types=[
            pltpu.VMEM_SHARED((_NN, 128), jnp.float32),
            pltpu.VMEM((128,), jnp.int32),
            pltpu.VMEM((128,), jnp.int32),
            pltpu.VMEM((tail,), jnp.int32),
            pltpu.VMEM((tail,), jnp.int32),
            pltpu.VMEM((128, 128), jnp.float32),
        ],
    )
    def k(g_hbm, src_hbm, dst_hbm, out_hbm, acc, si_v, di_v, st_v, dt_v, rows_v):
        c = lax.axis_index("c")
        s = lax.axis_index("s")
        gref = g_hbm.at[c] if fsplit else g_hbm

        rb = s * 640

        @pl.when(s < 15)
        def _():
            pltpu.sync_copy(gref.at[pl.ds(rb, 640)], acc.at[pl.ds(rb, 640)])

        @pl.when(s == 15)
        def _():
            pltpu.sync_copy(gref.at[pl.ds(9600, 400)], acc.at[pl.ds(9600, 400)])

        plsc.subcore_barrier()

        eb = s * 10000 if fsplit else c * 80000 + s * 5000

        @pl.loop(0, n_chunks)
        def _(j):
            b = eb + j * 128
            pltpu.sync_copy(src_hbm.at[pl.ds(b, 128)], si_v)
            pltpu.sync_copy(dst_hbm.at[pl.ds(b, 128)], di_v)
            pltpu.sync_copy(gref.at[si_v], rows_v)
            pltpu.sync_copy(rows_v, acc.at[di_v], add=True)

        tb = eb + n_chunks * 128
        pltpu.sync_copy(src_hbm.at[pl.ds(tb, tail)], st_v)
        pltpu.sync_copy(dst_hbm.at[pl.ds(tb, tail)], dt_v)
        pltpu.sync_copy(gref.at[st_v], rows_v.at[pl.ds(0, tail)])
        pltpu.sync_copy(rows_v.at[pl.ds(0, tail)], acc.at[dt_v], add=True)

        plsc.subcore_barrier()

        @pl.when(s < 15)
        def _():
            pltpu.sync_copy(acc.at[pl.ds(rb, 640)],
                            out_hbm.at[c, pl.ds(rb, 640)])

        @pl.when(s == 15)
        def _():
            pltpu.sync_copy(acc.at[pl.ds(9600, 400)],
                            out_hbm.at[c, pl.ds(9600, 400)])

    return k(g, src, dst)


def _tc_dinv(degp):
    def body(p_ref, o_ref):
        deg = jnp.max(p_ref[0], axis=1) + jnp.max(p_ref[1], axis=1) + 1.0
        o_ref[...] = lax.rsqrt(deg)

    return pl.pallas_call(
        body, out_shape=jax.ShapeDtypeStruct((_NP,), jnp.float32))(degp)


def _tc_layer1_in(x, W1, dinv):
    def body(x_ref, w_ref, dv_ref, o_ref):
        h = jnp.dot(x_ref[...], w_ref[...], preferred_element_type=jnp.float32)
        g = h * dv_ref[...][:, None]
        o_ref[0] = g[:, :128]
        o_ref[1] = g[:, 128:]

    return pl.pallas_call(
        body,
        grid=(10,),
        in_specs=[
            pl.BlockSpec((1024, 256), lambda i: (i, 0)),
            pl.BlockSpec((256, 256), lambda i: (0, 0)),
            pl.BlockSpec((1024,), lambda i: (i,)),
        ],
        out_specs=pl.BlockSpec((2, 1024, 128), lambda i: (0, i, 0)),
        out_shape=jax.ShapeDtypeStruct((2, _NN, 128), jnp.float32),
    )(x, W1, dinv)


def _tc_layer_mid(a1, dinv, b1, W2):
    def body(a_ref, dv_ref, b_ref, w_ref, o_ref):
        dv = dv_ref[...][:, None]
        t0 = a_ref[0] * dv + b_ref[...][None, :128]
        t1 = a_ref[1] * dv + b_ref[...][None, 128:]
        t = jnp.maximum(jnp.concatenate([t0, t1], axis=1), 0.0)
        o_ref[...] = jnp.dot(
            t, w_ref[...], preferred_element_type=jnp.float32) * dv

    return pl.pallas_call(
        body,
        grid=(10,),
        in_specs=[
            pl.BlockSpec((2, 1024, 128), lambda i: (0, i, 0)),
            pl.BlockSpec((1024,), lambda i: (i,)),
            pl.BlockSpec((256,), lambda i: (0,)),
            pl.BlockSpec((256, 128), lambda i: (0, 0)),
        ],
        out_specs=pl.BlockSpec((1024, 128), lambda i: (i, 0)),
        out_shape=jax.ShapeDtypeStruct((_NN, 128), jnp.float32),
    )(a1, dinv, b1, W2)


def _tc_layer2_out(a2, g2, dinv, b2):
    def body(p_ref, g_ref, dv_ref, b_ref, o_ref):
        agg = p_ref[0] + p_ref[1] - g_ref[...]
        o_ref[...] = agg * dv_ref[...][:, None] + b_ref[...][None, :]

    return pl.pallas_call(
        body,
        grid=(10,),
        in_specs=[
            pl.BlockSpec((2, 1024, 128), lambda i: (0, i, 0)),
            pl.BlockSpec((1024, 128), lambda i: (i, 0)),
            pl.BlockSpec((1024,), lambda i: (i,)),
            pl.BlockSpec((128,), lambda i: (0,)),
        ],
        out_specs=pl.BlockSpec((1024, 128), lambda i: (i, 0)),
        out_shape=jax.ShapeDtypeStruct((_NN, 128), jnp.float32),
    )(a2, g2, dinv, b2)


def kernel(x, edge_index, W1, b1, W2, b2):
    src = edge_index[0].astype(jnp.int32)
    dst = edge_index[1].astype(jnp.int32)

    zeros640 = jnp.zeros((640, 128), jnp.float32)
    ones128 = jnp.ones((128, 128), jnp.float32)
    degp = _sc_degree(dst, zeros640, ones128)
    dinv = _tc_dinv(degp)
    g1 = _tc_layer1_in(x, W1, dinv)
    a1 = _sc_agg(g1, src, dst, fsplit=True)
    g2 = _tc_layer_mid(a1, dinv, b1, W2)
    a2 = _sc_agg(g2, src, dst, fsplit=False)
    return _tc_layer2_out(a2, g2, dinv, b2)
